# baseline (speedup 1.0000x reference)
import functools

import jax
import jax.numpy as jnp
from jax.experimental import pallas as pl

N = 10000
E = 320000
F = 128
IC = 32
NB = F // IC
HC = 32
OUT = 128


def _matmul_body(x_ref, w_ref, b_ref, o_ref):
    o_ref[...] = jnp.dot(x_ref[...], w_ref[...],
                         preferred_element_type=jnp.float32) + b_ref[...]


def _tc_matmul(x, w, b, block_rows):
    n, f = x.shape
    _, m = w.shape
    grid = (n // block_rows,)
    return pl.pallas_call(
        _matmul_body,
        grid=grid,
        in_specs=[
            pl.BlockSpec((block_rows, f), lambda i: (i, 0)),
            pl.BlockSpec((f, m), lambda i: (0, 0)),
            pl.BlockSpec((1, m), lambda i: (0, 0)),
        ],
        out_specs=pl.BlockSpec((block_rows, m), lambda i: (i, 0)),
        out_shape=jax.ShapeDtypeStruct((n, m), jnp.float32),
    )(x, w, b)


def _block_diag(w):
    out = jnp.zeros((F, NB * HC), dtype=w.dtype)
    for i in range(NB):
        out = out.at[i * IC:(i + 1) * IC, i * HC:(i + 1) * HC].set(w)
    return out


def kernel(x, edge_index, edge_attrs, Wq, bq, Wk, bk, Wv, bv, We, Ws, bs, Wg, bg):
    src = edge_index[0]
    dst = edge_index[1]

    Wcat = jnp.concatenate(
        [_block_diag(Wq), _block_diag(Wk), _block_diag(Wv), _block_diag(Ws)],
        axis=1)
    bcat = jnp.concatenate(
        [jnp.tile(bq, NB), jnp.tile(bk, NB), jnp.tile(bv, NB), jnp.tile(bs, NB)])
    QKVS = _tc_matmul(x, Wcat, bcat[None, :], block_rows=2000)
    Q, K, V, S = (QKVS[:, 0:F], QKVS[:, F:2 * F], QKVS[:, 2 * F:3 * F],
                  QKVS[:, 3 * F:4 * F])

    e = edge_attrs @ We
    er = jnp.tile(e, (1, NB))

    ke = K[src] + er
    alpha = (Q[dst] * ke).reshape(E, NB, HC).sum(-1) / jnp.sqrt(jnp.float32(HC))
    ex = jnp.exp(alpha)
    den = jax.ops.segment_sum(ex, dst, num_segments=N)
    msg = (V[src] + er) * jnp.repeat(ex, HC, axis=1)
    num = jax.ops.segment_sum(msg, dst, num_segments=N)
    h = jax.nn.relu(num / jnp.repeat(den + 1e-16, HC, axis=1) + S)

    ew = edge_attrs[:, 1]
    deg = jax.ops.segment_sum(ew, dst, num_segments=N) + 2.0
    dis = deg ** -0.5
    norm = dis[src] * ew * dis[dst]
    hw = _tc_matmul(h, Wg, bg[None, :] * 0.0, block_rows=2000)
    out = jax.ops.segment_sum(norm[:, None] * hw[src], num_segments=N,
                              segment_ids=dst)
    out = out + (2.0 * dis * dis)[:, None] * hw + bg
    return out
